# baseline (speedup 1.0000x reference)
import jax
import jax.numpy as jnp
from jax.experimental import pallas as pl
from jax.experimental.pallas import tpu as pltpu

SEQ = 4096
FEAT = 26
DIM = 64
BATCH = 4


def _body(x_hbm, t_hbm, o_hbm, sem):
    pltpu.make_async_copy(x_hbm, o_hbm, sem).start()
    pltpu.make_async_copy(x_hbm, o_hbm, sem).wait()


def kernel(inputs, table):
    return pl.pallas_call(
        _body,
        grid=(1,),
        in_specs=[
            pl.BlockSpec(memory_space=pl.ANY),
            pl.BlockSpec(memory_space=pl.ANY),
        ],
        out_specs=pl.BlockSpec(memory_space=pl.ANY),
        out_shape=jax.ShapeDtypeStruct((BATCH, SEQ, FEAT, DIM), jnp.float32),
        scratch_shapes=[pltpu.SemaphoreType.DMA],
    )(inputs, table)


# speedup vs baseline: 10.8153x; 10.8153x over previous
import jax
import jax.numpy as jnp
from jax.experimental import pallas as pl
from jax.experimental.pallas import tpu as pltpu

SEQ = 4096
FEAT = 26
DIM = 64
BATCH = 4

BS = 64
NSTEP = SEQ // BS
DEPTH = 4


def _x_copy(x_hbm, xb, sx, step, slot, k):
    return pltpu.make_async_copy(
        x_hbm.at[k, pl.ds(step * BS, BS)], xb.at[slot, k], sx.at[slot, k]
    )


def _t_copy(t_hbm, tb, st, step, slot):
    return pltpu.make_async_copy(
        t_hbm.at[pl.ds(step * BS, BS)], tb.at[slot], st.at[slot]
    )


def _o_copy(o_hbm, ob, so, step, slot, k):
    return pltpu.make_async_copy(
        ob.at[slot, k], o_hbm.at[k, pl.ds(step * BS, BS)], so.at[slot, k]
    )


def _body(x_hbm, t_hbm, o_hbm, xb, tb, ob, sx, st, so):
    i = pl.program_id(0)
    slot = jax.lax.rem(i, DEPTH)

    def start_in(step, slot_):
        for k in range(BATCH):
            _x_copy(x_hbm, xb, sx, step, slot_, k).start()
        _t_copy(t_hbm, tb, st, step, slot_).start()

    @pl.when(i == 0)
    def _():
        for d in range(DEPTH - 1):
            start_in(d, d)

    @pl.when(i + DEPTH - 1 < NSTEP)
    def _():
        start_in(i + DEPTH - 1, jax.lax.rem(i + DEPTH - 1, DEPTH))

    for k in range(BATCH):
        _x_copy(x_hbm, xb, sx, i, slot, k).wait()
    _t_copy(t_hbm, tb, st, i, slot).wait()

    @pl.when(i >= DEPTH)
    def _():
        for k in range(BATCH):
            _o_copy(o_hbm, ob, so, i - DEPTH, slot, k).wait()

    t_val = tb[slot]
    for k in range(BATCH):
        ob[slot, k] = xb[slot, k] + t_val
    for k in range(BATCH):
        _o_copy(o_hbm, ob, so, i, slot, k).start()

    @pl.when(i == NSTEP - 1)
    def _():
        for d in range(DEPTH):
            s_ = jax.lax.rem(i - d, DEPTH)
            for k in range(BATCH):
                _o_copy(o_hbm, ob, so, i - d, s_, k).wait()


def kernel(inputs, table):
    t3 = table.reshape(SEQ, FEAT, DIM)
    return pl.pallas_call(
        _body,
        grid=(NSTEP,),
        in_specs=[
            pl.BlockSpec(memory_space=pl.ANY),
            pl.BlockSpec(memory_space=pl.ANY),
        ],
        out_specs=pl.BlockSpec(memory_space=pl.ANY),
        out_shape=jax.ShapeDtypeStruct((BATCH, SEQ, FEAT, DIM), jnp.float32),
        scratch_shapes=[
            pltpu.VMEM((DEPTH, BATCH, BS, FEAT, DIM), jnp.float32),
            pltpu.VMEM((DEPTH, BS, FEAT, DIM), jnp.float32),
            pltpu.VMEM((DEPTH, BATCH, BS, FEAT, DIM), jnp.float32),
            pltpu.SemaphoreType.DMA((DEPTH, BATCH)),
            pltpu.SemaphoreType.DMA((DEPTH,)),
            pltpu.SemaphoreType.DMA((DEPTH, BATCH)),
        ],
        compiler_params=pltpu.CompilerParams(
            dimension_semantics=("arbitrary",),
        ),
    )(inputs, t3)


# speedup vs baseline: 19.9735x; 1.8468x over previous
import jax
import jax.numpy as jnp
from jax.experimental import pallas as pl
from jax.experimental.pallas import tpu as pltpu

SEQ = 4096
FEAT = 26
DIM = 64
BATCH = 4
ROWD = FEAT * DIM

BS = 512


def _add_body(x_ref, t_ref, o_ref):
    o_ref[...] = x_ref[...] + t_ref[None]


def kernel(inputs, table):
    x = inputs.reshape(BATCH, SEQ, ROWD)
    t = table.reshape(SEQ, ROWD)
    out = pl.pallas_call(
        _add_body,
        grid=(SEQ // BS, BATCH),
        in_specs=[
            pl.BlockSpec((1, BS, ROWD), lambda s, b: (b, s, 0)),
            pl.BlockSpec((BS, ROWD), lambda s, b: (s, 0)),
        ],
        out_specs=pl.BlockSpec((1, BS, ROWD), lambda s, b: (b, s, 0)),
        out_shape=jax.ShapeDtypeStruct((BATCH, SEQ, ROWD), jnp.float32),
        compiler_params=pltpu.CompilerParams(
            dimension_semantics=("arbitrary", "arbitrary"),
        ),
    )(x, t)
    return out.reshape(BATCH, SEQ, FEAT, DIM)


# speedup vs baseline: 20.2225x; 1.0125x over previous
import jax
import jax.numpy as jnp
from jax.experimental import pallas as pl
from jax.experimental.pallas import tpu as pltpu

SEQ = 4096
FEAT = 26
DIM = 64
BATCH = 4
ROWD = FEAT * DIM

BS = 1024


def _add_body(x_ref, t_ref, o_ref):
    o_ref[...] = x_ref[...] + t_ref[None]


def kernel(inputs, table):
    x = inputs.reshape(BATCH, SEQ, ROWD)
    t = table.reshape(SEQ, ROWD)
    out = pl.pallas_call(
        _add_body,
        grid=(SEQ // BS, BATCH),
        in_specs=[
            pl.BlockSpec((1, BS, ROWD), lambda s, b: (b, s, 0)),
            pl.BlockSpec((BS, ROWD), lambda s, b: (s, 0)),
        ],
        out_specs=pl.BlockSpec((1, BS, ROWD), lambda s, b: (b, s, 0)),
        out_shape=jax.ShapeDtypeStruct((BATCH, SEQ, ROWD), jnp.float32),
        compiler_params=pltpu.CompilerParams(
            dimension_semantics=("arbitrary", "arbitrary"),
        ),
    )(x, t)
    return out.reshape(BATCH, SEQ, FEAT, DIM)


# speedup vs baseline: 20.2866x; 1.0032x over previous
import jax
import jax.numpy as jnp
from jax.experimental import pallas as pl
from jax.experimental.pallas import tpu as pltpu

SEQ = 4096
FEAT = 26
DIM = 64
BATCH = 4
ROWD = FEAT * DIM

CH = 128
NSTEP = SEQ // CH
DEPTH = 6


def _x_copy(x_hbm, xb, sx, step, slot, k):
    return pltpu.make_async_copy(
        x_hbm.at[k, pl.ds(step * CH, CH)], xb.at[slot, k], sx.at[slot, k]
    )


def _t_copy(t_hbm, tb, st, step, slot):
    return pltpu.make_async_copy(
        t_hbm.at[pl.ds(step * CH, CH)], tb.at[slot], st.at[slot]
    )


def _o_copy(o_hbm, ob, so, step, slot, k):
    return pltpu.make_async_copy(
        ob.at[slot, k], o_hbm.at[k, pl.ds(step * CH, CH)], so.at[slot, k]
    )


def _body(x_hbm, t_hbm, o_hbm, xb, tb, ob, sx, st, so):
    i = pl.program_id(0)
    slot = jax.lax.rem(i, DEPTH)

    def start_in(step, slot_):
        for k in range(BATCH):
            _x_copy(x_hbm, xb, sx, step, slot_, k).start()
        _t_copy(t_hbm, tb, st, step, slot_).start()

    @pl.when(i == 0)
    def _():
        for d in range(DEPTH - 1):
            start_in(d, d)

    @pl.when(i + DEPTH - 1 < NSTEP)
    def _():
        start_in(i + DEPTH - 1, jax.lax.rem(i + DEPTH - 1, DEPTH))

    for k in range(BATCH):
        _x_copy(x_hbm, xb, sx, i, slot, k).wait()
    _t_copy(t_hbm, tb, st, i, slot).wait()

    @pl.when(i >= DEPTH)
    def _():
        for k in range(BATCH):
            _o_copy(o_hbm, ob, so, i - DEPTH, slot, k).wait()

    t_val = tb[slot]
    for k in range(BATCH):
        ob[slot, k] = xb[slot, k] + t_val
    for k in range(BATCH):
        _o_copy(o_hbm, ob, so, i, slot, k).start()

    @pl.when(i == NSTEP - 1)
    def _():
        for d in range(DEPTH):
            s_ = jax.lax.rem(i - d, DEPTH)
            for k in range(BATCH):
                _o_copy(o_hbm, ob, so, i - d, s_, k).wait()


def kernel(inputs, table):
    x = inputs.reshape(BATCH, SEQ, ROWD)
    t = table.reshape(SEQ, ROWD)
    out = pl.pallas_call(
        _body,
        grid=(NSTEP,),
        in_specs=[
            pl.BlockSpec(memory_space=pl.ANY),
            pl.BlockSpec(memory_space=pl.ANY),
        ],
        out_specs=pl.BlockSpec(memory_space=pl.ANY),
        out_shape=jax.ShapeDtypeStruct((BATCH, SEQ, ROWD), jnp.float32),
        scratch_shapes=[
            pltpu.VMEM((DEPTH, BATCH, CH, ROWD), jnp.float32),
            pltpu.VMEM((DEPTH, CH, ROWD), jnp.float32),
            pltpu.VMEM((DEPTH, BATCH, CH, ROWD), jnp.float32),
            pltpu.SemaphoreType.DMA((DEPTH, BATCH)),
            pltpu.SemaphoreType.DMA((DEPTH,)),
            pltpu.SemaphoreType.DMA((DEPTH, BATCH)),
        ],
        compiler_params=pltpu.CompilerParams(
            dimension_semantics=("arbitrary",),
        ),
    )(x, t)
    return out.reshape(BATCH, SEQ, FEAT, DIM)
